# baseline (speedup 1.0000x reference)
import functools

import jax
import jax.numpy as jnp
from jax import lax
from jax.experimental import pallas as pl
from jax.experimental.pallas import tpu as pltpu
from jax.experimental.pallas import tpu_sc as plsc

_NC = 2


@functools.lru_cache(maxsize=None)
def _make_sc_broadcast(B, V):
    rows_sc = B // _NC
    R = 32
    n_out = rows_sc // R

    mesh = plsc.ScalarSubcoreMesh(axis_name="c")

    @functools.partial(
        pl.kernel,
        out_type=jax.ShapeDtypeStruct((B, V), jnp.float32),
        mesh=mesh,
        scratch_types=[
            pltpu.VMEM_SHARED((R, V), jnp.float32),
            pltpu.SemaphoreType.DMA,
        ],
    )
    def broadcast_kernel(table_hbm, out_hbm, shared_buf, sem):
        cid = lax.axis_index("c")
        fills = [
            pltpu.async_copy(table_hbm, shared_buf.at[r], sem)
            for r in range(R)
        ]
        for cp in fills:
            cp.wait()
        base = cid * rows_sc
        copies = [
            pltpu.async_copy(
                shared_buf,
                out_hbm.at[pl.ds(base + c * R, R)],
                sem,
            )
            for c in range(n_out)
        ]
        for cp in copies:
            cp.wait()

    return broadcast_kernel


def kernel(x, table):
    B = x.shape[0]
    V = table.shape[0]
    fn = _make_sc_broadcast(B, V)
    return fn(table.reshape(V))


# speedup vs baseline: 1.0088x; 1.0088x over previous
import functools

import jax
import jax.numpy as jnp
from jax import lax
from jax.experimental import pallas as pl
from jax.experimental.pallas import tpu as pltpu
from jax.experimental.pallas import tpu_sc as plsc

_NC = 2


@functools.lru_cache(maxsize=None)
def _make_sc_broadcast(B, V):
    rows_sc = B // _NC
    R = 8
    n_out = rows_sc // R

    mesh = plsc.ScalarSubcoreMesh(axis_name="c")

    @functools.partial(
        pl.kernel,
        out_type=jax.ShapeDtypeStruct((B, V), jnp.float32),
        mesh=mesh,
        scratch_types=[
            pltpu.VMEM_SHARED((R, V), jnp.float32),
            pltpu.SemaphoreType.DMA,
        ],
    )
    def broadcast_kernel(table_hbm, out_hbm, shared_buf, sem):
        cid = lax.axis_index("c")
        fills = [
            pltpu.async_copy(table_hbm, shared_buf.at[r], sem)
            for r in range(R)
        ]
        for cp in fills:
            cp.wait()
        base = cid * rows_sc
        copies = [
            pltpu.async_copy(
                shared_buf,
                out_hbm.at[pl.ds(base + c * R, R)],
                sem,
            )
            for c in range(n_out)
        ]
        for cp in copies:
            cp.wait()

    return broadcast_kernel


def kernel(x, table):
    B = x.shape[0]
    V = table.shape[0]
    fn = _make_sc_broadcast(B, V)
    return fn(table.reshape(V))


# speedup vs baseline: 1.0182x; 1.0092x over previous
import functools

import jax
import jax.numpy as jnp
from jax import lax
from jax.experimental import pallas as pl
from jax.experimental.pallas import tpu as pltpu
from jax.experimental.pallas import tpu_sc as plsc

_NC = 2


@functools.lru_cache(maxsize=None)
def _make_sc_broadcast(B, V):
    rows_sc = B // _NC
    R = 16
    n_out = rows_sc // R

    mesh = plsc.ScalarSubcoreMesh(axis_name="c")

    @functools.partial(
        pl.kernel,
        out_type=jax.ShapeDtypeStruct((B, V), jnp.float32),
        mesh=mesh,
        scratch_types=[
            pltpu.VMEM_SHARED((R, V), jnp.float32),
            pltpu.SemaphoreType.DMA,
        ],
    )
    def broadcast_kernel(table_hbm, out_hbm, shared_buf, sem):
        cid = lax.axis_index("c")
        fills = [
            pltpu.async_copy(table_hbm, shared_buf.at[r], sem)
            for r in range(R)
        ]
        for cp in fills:
            cp.wait()
        base = cid * rows_sc
        copies = [
            pltpu.async_copy(
                shared_buf,
                out_hbm.at[pl.ds(base + c * R, R)],
                sem,
            )
            for c in range(n_out)
        ]
        for cp in copies:
            cp.wait()

    return broadcast_kernel


def kernel(x, table):
    B = x.shape[0]
    V = table.shape[0]
    fn = _make_sc_broadcast(B, V)
    return fn(table.reshape(V))
